# baseline (speedup 1.0000x reference)
import jax
import jax.numpy as jnp
from jax import lax
from jax.experimental import pallas as pl
from jax.experimental.pallas import tpu as pltpu
from jax.experimental.pallas import tpu_sc as plsc

BATCH = 16384
DIM = 2048
LANES = 16
GROUPS = DIM // LANES

_info = plsc.get_sparse_core_info()
NUM_CORES = _info.num_cores
NUM_SUBCORES = _info.num_subcores
NUM_WORKERS = NUM_CORES * NUM_SUBCORES
ROWS_PER_WORKER = BATCH // NUM_WORKERS
BLOCK_ROWS = 4
NUM_BLOCKS = ROWS_PER_WORKER // BLOCK_ROWS
BLOCK_ELEMS = BLOCK_ROWS * DIM
NBUF = 4
NUM_PHASES = NUM_BLOCKS // NBUF
UNROLL = 4


def _sc_body(
    z_hbm, perm_hbm, out_hbm, perm_v, in_bufs, out_bufs, in_sems, out_sems
):
    wid = lax.axis_index("s") * NUM_CORES + lax.axis_index("c")

    pltpu.sync_copy(perm_hbm, perm_v)

    def _off(g):
        return (g * NUM_WORKERS + wid) * BLOCK_ELEMS

    def issue_fetch(g, b):
        pltpu.async_copy(
            z_hbm.at[pl.ds(_off(g), BLOCK_ELEMS)], in_bufs[b], in_sems[b]
        )

    def issue_store(g, b):
        pltpu.async_copy(
            out_bufs[b], out_hbm.at[pl.ds(_off(g), BLOCK_ELEMS)], out_sems[b]
        )

    def wait_fetch(b):
        pltpu.make_async_copy(
            z_hbm.at[pl.ds(0, BLOCK_ELEMS)], in_bufs[b], in_sems[b]
        ).wait()

    def wait_store(b):
        pltpu.make_async_copy(
            out_bufs[b], out_hbm.at[pl.ds(0, BLOCK_ELEMS)], out_sems[b]
        ).wait()

    def gather_block(b):
        in_buf = in_bufs[b]
        out_buf = out_bufs[b]

        @plsc.parallel_loop(0, GROUPS, unroll=UNROLL)
        def _(j):
            idx16 = perm_v[pl.ds(j * LANES, LANES)]
            for r in range(BLOCK_ROWS):
                vals = plsc.load_gather(in_buf, [idx16 + (r * DIM)])
                out_buf[pl.ds(r * DIM + j * LANES, LANES)] = vals

    for b in range(NBUF):
        issue_fetch(b, b)
    for b in range(NBUF):
        wait_fetch(b)
        gather_block(b)
        issue_store(b, b)
        issue_fetch(b + NBUF, b)

    def phase_step(p, _):
        for b in range(NBUF):
            g = p * NBUF + b
            wait_fetch(b)
            wait_store(b)
            gather_block(b)
            issue_store(g, b)

            @pl.when(g + NBUF < NUM_BLOCKS)
            def _():
                issue_fetch(g + NBUF, b)

        return 0

    lax.fori_loop(1, NUM_PHASES, phase_step, 0)

    for b in range(NBUF):
        wait_store(b)


@jax.jit
def _permute(z_flat, perm):
    mesh = plsc.VectorSubcoreMesh(core_axis_name="c", subcore_axis_name="s")
    return pl.kernel(
        _sc_body,
        out_type=jax.ShapeDtypeStruct((BATCH * DIM,), jnp.float32),
        mesh=mesh,
        compiler_params=pltpu.CompilerParams(needs_layout_passes=False),
        scratch_types=[
            pltpu.VMEM((DIM,), jnp.int32),
            [pltpu.VMEM((BLOCK_ELEMS,), jnp.float32) for _ in range(NBUF)],
            [pltpu.VMEM((BLOCK_ELEMS,), jnp.float32) for _ in range(NBUF)],
            [pltpu.SemaphoreType.DMA for _ in range(NBUF)],
            [pltpu.SemaphoreType.DMA for _ in range(NBUF)],
        ],
    )(z_flat, perm)


def kernel(z, permutation):
    out = _permute(z.reshape(-1), permutation.astype(jnp.int32))
    return out.reshape(BATCH, DIM)
